# Initial kernel scaffold; baseline (speedup 1.0000x reference)
import jax
import jax.numpy as jnp
from jax.experimental import pallas as pl


def kernel(embedding, edge_index, W1, b1, W2, b2, W3, b3, W4, b4, Wc, bc):
    raise NotImplementedError("write your pallas kernel here")


# speedup vs baseline: 3.4960x; 3.4960x over previous
import functools

import jax
import jax.numpy as jnp
from jax import lax
from jax.experimental import pallas as pl
from jax.experimental.pallas import tpu as pltpu
from jax.experimental.pallas import tpu_sc as plsc

N_NODES = 10000
N_EDGES = 320000
D = 128
C = 16

NACC = 10240
NCORES = 2
NSUB = 16
NW = NCORES * NSUB
CHUNK = 128
EPW = 10112
NCHUNKS = EPW // CHUNK
EPAD = EPW * NW
ROWS_PER_TILE = NACC // NSUB


def _hist_body(src_hbm, dst_hbm, ones_hbm, zeros_hbm,
               dego_out, degi_out, acc_sh, idx, ones_v):
    cid = lax.axis_index("c")
    sid = lax.axis_index("s")
    wid = cid * NSUB + sid
    base = wid * EPW
    row0 = sid * ROWS_PER_TILE

    pltpu.sync_copy(ones_hbm, ones_v)

    for idx_hbm, out_hbm in ((src_hbm, dego_out), (dst_hbm, degi_out)):
        pltpu.sync_copy(zeros_hbm, acc_sh.at[pl.ds(row0, ROWS_PER_TILE)])
        plsc.subcore_barrier()

        def body(i, carry):
            off = base + i * CHUNK
            pltpu.sync_copy(idx_hbm.at[pl.ds(off, CHUNK)], idx)
            pltpu.sync_copy(ones_v, acc_sh.at[idx], add=True)
            return carry

        lax.fori_loop(0, NCHUNKS, body, 0)
        plsc.subcore_barrier()

        pltpu.sync_copy(acc_sh.at[pl.ds(row0, ROWS_PER_TILE)],
                        out_hbm.at[cid, pl.ds(row0, ROWS_PER_TILE)])


def _agg_body(h_hbm, src_hbm, dst_hbm, zeros_hbm, out_hbm,
              acc_sh, sidx, didx, rows, gsem):
    cid = lax.axis_index("c")
    sid = lax.axis_index("s")
    wid = cid * NSUB + sid
    base = wid * EPW
    row0 = sid * ROWS_PER_TILE

    pltpu.sync_copy(zeros_hbm, acc_sh.at[pl.ds(row0, ROWS_PER_TILE)])
    plsc.subcore_barrier()

    def body(i, carry):
        off = base + i * CHUNK
        pltpu.sync_copy(src_hbm.at[pl.ds(off, CHUNK)], sidx)
        pltpu.sync_copy(dst_hbm.at[pl.ds(off, CHUNK)], didx)
        pltpu.async_copy(h_hbm.at[sidx], rows, gsem).wait()
        pltpu.sync_copy(rows, acc_sh.at[didx], add=True)
        return carry

    lax.fori_loop(0, NCHUNKS, body, 0)
    plsc.subcore_barrier()

    pltpu.sync_copy(acc_sh.at[pl.ds(row0, ROWS_PER_TILE)],
                    out_hbm.at[cid, pl.ds(row0, ROWS_PER_TILE)])


@functools.cache
def _sc_kernels():
    mesh = plsc.VectorSubcoreMesh(core_axis_name="c", subcore_axis_name="s",
                                  num_cores=NCORES, num_subcores=NSUB)
    deg = pl.kernel(
        _hist_body,
        out_type=[
            jax.ShapeDtypeStruct((NCORES, NACC, D), jnp.float32),
            jax.ShapeDtypeStruct((NCORES, NACC, D), jnp.float32),
        ],
        mesh=mesh,
        scratch_types=[
            pltpu.VMEM_SHARED((NACC, D), jnp.float32),
            pltpu.VMEM((CHUNK,), jnp.int32),
            pltpu.VMEM((CHUNK, D), jnp.float32),
        ],
    )
    agg = pl.kernel(
        _agg_body,
        out_type=jax.ShapeDtypeStruct((NCORES, NACC, D), jnp.float32),
        mesh=mesh,
        scratch_types=[
            pltpu.VMEM_SHARED((NACC, D), jnp.float32),
            pltpu.VMEM((CHUNK,), jnp.int32),
            pltpu.VMEM((CHUNK,), jnp.int32),
            pltpu.VMEM((CHUNK, D), jnp.float32),
            pltpu.SemaphoreType.DMA,
        ],
    )
    return deg, agg


BLK = 512
NBLK = NACC // BLK


def _row_mask(pid):
    rows = pid * BLK + lax.broadcasted_iota(jnp.int32, (BLK, 1), 0)
    return rows < N_NODES


def _normprep_body(dego_ref, degi_ref, emb_ref, h_ref, ns_ref, nd_ref):
    do3 = dego_ref[...]
    di3 = degi_ref[...]
    do = jnp.sum(do3[0] + do3[1], axis=1, keepdims=True) * (1.0 / D)
    di = jnp.sum(di3[0] + di3[1], axis=1, keepdims=True) * (1.0 / D)
    ns = lax.rsqrt(jnp.maximum(do, 1.0))
    nd = lax.rsqrt(jnp.maximum(di, 1.0))
    ns_ref[...] = ns
    nd_ref[...] = nd
    h_ref[...] = emb_ref[...] * ns


_normprep = pl.pallas_call(
    _normprep_body,
    grid=(NBLK,),
    in_specs=[
        pl.BlockSpec((NCORES, BLK, D), lambda i: (0, i, 0)),
        pl.BlockSpec((NCORES, BLK, D), lambda i: (0, i, 0)),
        pl.BlockSpec((BLK, D), lambda i: (i, 0)),
    ],
    out_specs=[
        pl.BlockSpec((BLK, D), lambda i: (i, 0)),
        pl.BlockSpec((BLK, 1), lambda i: (i, 0)),
        pl.BlockSpec((BLK, 1), lambda i: (i, 0)),
    ],
    out_shape=[
        jax.ShapeDtypeStruct((NACC, D), jnp.float32),
        jax.ShapeDtypeStruct((NACC, 1), jnp.float32),
        jax.ShapeDtypeStruct((NACC, 1), jnp.float32),
    ],
)


def _dense_body(acc_ref, nd_ref, ns_ref, w_ref, b_ref, out_ref):
    pid = pl.program_id(0)
    a3 = acc_ref[...]
    agg = a3[0] + a3[1]
    t = jnp.dot(agg, w_ref[...], preferred_element_type=jnp.float32)
    y = jnp.maximum(t * nd_ref[...] + b_ref[...], 0.0)
    y = jnp.where(_row_mask(pid), y, 0.0)
    out_ref[...] = y * ns_ref[...]


_dense = pl.pallas_call(
    _dense_body,
    grid=(NBLK,),
    in_specs=[
        pl.BlockSpec((NCORES, BLK, D), lambda i: (0, i, 0)),
        pl.BlockSpec((BLK, 1), lambda i: (i, 0)),
        pl.BlockSpec((BLK, 1), lambda i: (i, 0)),
        pl.BlockSpec((D, D), lambda i: (0, 0)),
        pl.BlockSpec((1, D), lambda i: (0, 0)),
    ],
    out_specs=pl.BlockSpec((BLK, D), lambda i: (i, 0)),
    out_shape=jax.ShapeDtypeStruct((NACC, D), jnp.float32),
)


def _final_body(acc_ref, nd_ref, w_ref, b_ref, wc_ref, bc_ref,
                logits_ref, hg_ref):
    pid = pl.program_id(0)
    a3 = acc_ref[...]
    agg = a3[0] + a3[1]
    t = jnp.dot(agg, w_ref[...], preferred_element_type=jnp.float32)
    y = jnp.maximum(t * nd_ref[...] + b_ref[...], 0.0)
    y = jnp.where(_row_mask(pid), y, 0.0)
    s = jnp.sum(y, axis=0, keepdims=True)

    @pl.when(pid == 0)
    def _():
        hg_ref[...] = jnp.zeros_like(hg_ref)

    hg_ref[...] += s

    @pl.when(pid == NBLK - 1)
    def _():
        hgm = hg_ref[...] * (1.0 / N_NODES)
        hg_ref[...] = hgm
        logits_ref[...] = (
            jnp.dot(hgm, wc_ref[...], preferred_element_type=jnp.float32)
            + bc_ref[...])


_final = pl.pallas_call(
    _final_body,
    grid=(NBLK,),
    in_specs=[
        pl.BlockSpec((NCORES, BLK, D), lambda i: (0, i, 0)),
        pl.BlockSpec((BLK, 1), lambda i: (i, 0)),
        pl.BlockSpec((D, D), lambda i: (0, 0)),
        pl.BlockSpec((1, D), lambda i: (0, 0)),
        pl.BlockSpec((D, C), lambda i: (0, 0)),
        pl.BlockSpec((1, C), lambda i: (0, 0)),
    ],
    out_specs=[
        pl.BlockSpec((1, C), lambda i: (0, 0)),
        pl.BlockSpec((1, D), lambda i: (0, 0)),
    ],
    out_shape=[
        jax.ShapeDtypeStruct((1, C), jnp.float32),
        jax.ShapeDtypeStruct((1, D), jnp.float32),
    ],
)


def kernel(embedding, edge_index, W1, b1, W2, b2, W3, b3, W4, b4, Wc, bc):
    src = edge_index[0].astype(jnp.int32)
    dst = edge_index[1].astype(jnp.int32)
    pad = EPAD - N_EDGES
    src_p = jnp.concatenate([src, jnp.full((pad,), N_NODES, jnp.int32)])
    dst_p = jnp.concatenate([dst, jnp.full((pad,), N_NODES, jnp.int32)])
    emb_pad = jnp.pad(embedding, ((0, NACC - N_NODES), (0, 0)))

    zeros128 = jnp.zeros((ROWS_PER_TILE, D), jnp.float32)
    ones128 = jnp.ones((CHUNK, D), jnp.float32)

    deg_kernel, agg_kernel = _sc_kernels()
    dego, degi = deg_kernel(src_p, dst_p, ones128, zeros128)
    h, ns, nd = _normprep(dego, degi, emb_pad)

    for W, b in ((W1, b1), (W2, b2), (W3, b3)):
        acc = agg_kernel(h, src_p, dst_p, zeros128)
        h = _dense(acc, nd, ns, W, b.reshape(1, D))

    acc = agg_kernel(h, src_p, dst_p, zeros128)
    logits, hg = _final(acc, nd, W4, b4.reshape(1, D), Wc, bc.reshape(1, C))
    return (logits, hg)
